# baseline (speedup 1.0000x reference)
import functools

import jax
import jax.numpy as jnp
from jax import lax
from jax.experimental import pallas as pl
from jax.experimental.pallas import tpu as pltpu
from jax.experimental.pallas import tpu_sc as plsc

_NC = 2
_NS = 16
_NW = _NC * _NS
_LANES = 16
_CHUNK = 128


_GATHER_DNUMS = lax.GatherDimensionNumbers(
    offset_dims=(), collapsed_slice_dims=(0,), start_index_map=(0,))


def _lane_broadcast(v, l):
    ids = jnp.full((_LANES, 1), l, jnp.int32)
    return lax.gather(v, ids, _GATHER_DNUMS, slice_sizes=(1,),
                      mode=lax.GatherScatterMode.PROMISE_IN_BOUNDS)


def _neg_sqrt(s):
    bits = plsc.bitcast(s, jnp.int32)
    r = plsc.bitcast(jnp.int32(0x5F3759DF) - (bits >> 1), jnp.float32)
    half_s = 0.5 * s
    for _ in range(3):
        r = r * (1.5 - half_s * r * r)
    return -(s * r)


def _make_sc_kernel(batch, dim, chunks):
    b_per_w = chunks * _CHUNK
    groups = _CHUNK // _LANES

    mesh = plsc.VectorSubcoreMesh(core_axis_name="c", subcore_axis_name="s")

    @functools.partial(
        pl.kernel,
        mesh=mesh,
        compiler_params=pltpu.CompilerParams(
            needs_layout_passes=False, use_tc_tiling_on_sc=True),
        out_type=jax.ShapeDtypeStruct((batch,), jnp.float32),
        scratch_types=[
            pltpu.VMEM((chunks, _CHUNK), jnp.int32),
            pltpu.VMEM((chunks, _CHUNK), jnp.int32),
            pltpu.VMEM((chunks, _CHUNK), jnp.int32),
            pltpu.VMEM((chunks, _CHUNK), jnp.int32),
            pltpu.VMEM((chunks, _CHUNK), jnp.int32),
            pltpu.VMEM((chunks, _CHUNK), jnp.int32),
            pltpu.VMEM((_CHUNK, 2 * dim), jnp.float32),
            pltpu.VMEM((_CHUNK, 2 * dim), jnp.float32),
            pltpu.VMEM((_CHUNK, 2 * dim), jnp.float32),
            pltpu.VMEM((_CHUNK, 2 * dim), jnp.float32),
            pltpu.VMEM((_CHUNK, 2 * dim), jnp.float32),
            pltpu.VMEM((_CHUNK, 2 * dim), jnp.float32),
            pltpu.VMEM((b_per_w,), jnp.float32),
            pltpu.SemaphoreType.DMA,
            pltpu.SemaphoreType.DMA,
        ],
    )
    def sc_kernel(heads_hbm, rels_hbm, tails_hbm, ph_hbm, pr_hbm, pt_hbm,
                  ent_hbm, rel_hbm, out_hbm,
                  idx_h, idx_r, idx_t, par_h, par_r, par_t,
                  hb0, hb1, rb0, rb1, tb0, tb1, outv, sem0, sem1):
        wid = lax.axis_index("s") * _NC + lax.axis_index("c")

        pltpu.sync_copy(heads_hbm.at[wid], idx_h)
        pltpu.sync_copy(rels_hbm.at[wid], idx_r)
        pltpu.sync_copy(tails_hbm.at[wid], idx_t)
        pltpu.sync_copy(ph_hbm.at[wid], par_h)
        pltpu.sync_copy(pr_hbm.at[wid], par_r)
        pltpu.sync_copy(pt_hbm.at[wid], par_t)

        hb = (hb0, hb1)
        rb = (rb0, rb1)
        tb = (tb0, tb1)
        sems = (sem0, sem1)

        def fire(c):
            slot = c % 2
            return (
                pltpu.async_copy(ent_hbm.at[idx_h.at[c]], hb[slot], sems[slot]),
                pltpu.async_copy(rel_hbm.at[idx_r.at[c]], rb[slot], sems[slot]),
                pltpu.async_copy(ent_hbm.at[idx_t.at[c]], tb[slot], sems[slot]),
            )

        vregs_per_half = dim // _LANES
        lane = lax.broadcasted_iota(jnp.int32, (_LANES,), 0)

        def compute_chunk(c):
            slot = c % 2
            hbuf, rbuf, tbuf = hb[slot], rb[slot], tb[slot]

            def group_body(g, carry):
                base = g * _LANES
                gsl = pl.ds(base, _LANES)
                pvh = par_h[c, gsl]
                pvr = par_r[c, gsl]
                pvt = par_t[c, gsl]

                def triple_body(l, s):
                    lrow = base + l
                    mh = _lane_broadcast(pvh, l) > 0
                    mr = _lane_broadcast(pvr, l) > 0
                    mt = _lane_broadcast(pvt, l) > 0
                    acc = None
                    for j in range(vregs_per_half):
                        lo = pl.ds(j * _LANES, _LANES)
                        hi = pl.ds(dim + j * _LANES, _LANES)
                        hv = jnp.where(mh, hbuf[lrow, hi], hbuf[lrow, lo])
                        rv = jnp.where(mr, rbuf[lrow, hi], rbuf[lrow, lo])
                        tv = jnp.where(mt, tbuf[lrow, hi], tbuf[lrow, lo])
                        d = hv + rv - tv
                        sq = d * d
                        acc = sq if acc is None else acc + sq
                    return jnp.where(lane == l, jnp.sum(acc), s)

                s = lax.fori_loop(0, _LANES, triple_body,
                                  jnp.zeros((_LANES,), jnp.float32))
                outv[pl.ds(c * _CHUNK + base, _LANES)] = _neg_sqrt(s)
                return carry

            lax.fori_loop(0, groups, group_body, 0)

        descs = fire(0)
        for c in range(chunks):
            nxt = fire(c + 1) if c + 1 < chunks else None
            for d in descs:
                d.wait()
            compute_chunk(c)
            descs = nxt

        pltpu.sync_copy(outv, out_hbm.at[pl.ds(wid * b_per_w, b_per_w)])

    return sc_kernel


def _repack_block(tin_ref, out_ref):
    dim = tin_ref.shape[0]
    bc = tin_ref.shape[1] // 2
    t = tin_ref[...].T
    out_ref[:, 0:dim] = t[0:bc, :]
    out_ref[:, dim:2 * dim] = t[bc:2 * bc, :]


def _repack(table_t, bc):
    dim, n = table_t.shape
    nb = -(-n // (2 * bc))
    return pl.pallas_call(
        _repack_block,
        grid=(nb,),
        in_specs=[pl.BlockSpec((dim, 2 * bc), lambda c: (0, c))],
        out_specs=pl.BlockSpec((bc, 2 * dim), lambda c: (c, 0)),
        out_shape=jax.ShapeDtypeStruct((nb * bc, 2 * dim), jnp.float32),
    )(table_t)


_BC = 512


def kernel(heads, relations, tails, entity_emb, relation_emb):
    batch = heads.shape[0]
    dim = entity_emb.shape[1]
    chunks = batch // (_NW * _CHUNK)

    ent2 = _repack(entity_emb.T, _BC)
    rel2 = _repack(relation_emb.T, _BC)

    def split(ix):
        ix = ix.astype(jnp.int32).reshape(_NW, chunks, _CHUNK)
        return ((ix >> 10) << 9) + (ix & (_BC - 1)), (ix >> 9) & 1

    hh, ph = split(heads)
    rh, pr = split(relations)
    th, pt = split(tails)

    out = _make_sc_kernel(batch, dim, chunks)(
        hh, rh, th, ph, pr, pt, ent2, rel2)
    return out.reshape(batch, 1)


# speedup vs baseline: 2.7042x; 2.7042x over previous
import functools

import jax
import jax.numpy as jnp
from jax import lax
from jax.experimental import pallas as pl
from jax.experimental.pallas import tpu as pltpu
from jax.experimental.pallas import tpu_sc as plsc

_NC = 2
_NS = 16
_NW = _NC * _NS
_LANES = 16
_CHUNK = 128


_GATHER_DNUMS = lax.GatherDimensionNumbers(
    offset_dims=(), collapsed_slice_dims=(0,), start_index_map=(0,))


def _lane_broadcast(v, l):
    ids = jnp.full((_LANES, 1), l, jnp.int32)
    return lax.gather(v, ids, _GATHER_DNUMS, slice_sizes=(1,),
                      mode=lax.GatherScatterMode.PROMISE_IN_BOUNDS)


def _neg_sqrt(s):
    bits = plsc.bitcast(s, jnp.int32)
    r = plsc.bitcast(jnp.int32(0x5F3759DF) - (bits >> 1), jnp.float32)
    half_s = 0.5 * s
    for _ in range(3):
        r = r * (1.5 - half_s * r * r)
    return -(s * r)


def _make_sc_kernel(batch, dim, chunks):
    b_per_w = chunks * _CHUNK
    groups = _CHUNK // _LANES

    mesh = plsc.VectorSubcoreMesh(core_axis_name="c", subcore_axis_name="s")

    @functools.partial(
        pl.kernel,
        mesh=mesh,
        compiler_params=pltpu.CompilerParams(
            needs_layout_passes=False, use_tc_tiling_on_sc=True),
        out_type=jax.ShapeDtypeStruct((batch,), jnp.float32),
        scratch_types=[
            pltpu.VMEM((chunks, _CHUNK), jnp.int32),
            pltpu.VMEM((chunks, _CHUNK), jnp.int32),
            pltpu.VMEM((chunks, _CHUNK), jnp.int32),
            pltpu.VMEM((chunks, _CHUNK), jnp.int32),
            pltpu.VMEM((chunks, _CHUNK), jnp.int32),
            pltpu.VMEM((chunks, _CHUNK), jnp.int32),
            pltpu.VMEM((_CHUNK, 2 * dim), jnp.float32),
            pltpu.VMEM((_CHUNK, 2 * dim), jnp.float32),
            pltpu.VMEM((_CHUNK, 2 * dim), jnp.float32),
            pltpu.VMEM((_CHUNK, 2 * dim), jnp.float32),
            pltpu.VMEM((_CHUNK, 2 * dim), jnp.float32),
            pltpu.VMEM((_CHUNK, 2 * dim), jnp.float32),
            pltpu.VMEM((b_per_w,), jnp.float32),
            pltpu.SemaphoreType.DMA,
            pltpu.SemaphoreType.DMA,
        ],
    )
    def sc_kernel(heads_hbm, rels_hbm, tails_hbm, ph_hbm, pr_hbm, pt_hbm,
                  ent_hbm, rel_hbm, out_hbm,
                  idx_h, idx_r, idx_t, par_h, par_r, par_t,
                  hb0, hb1, rb0, rb1, tb0, tb1, outv, sem0, sem1):
        wid = lax.axis_index("s") * _NC + lax.axis_index("c")

        pltpu.sync_copy(heads_hbm.at[wid], idx_h)
        pltpu.sync_copy(rels_hbm.at[wid], idx_r)
        pltpu.sync_copy(tails_hbm.at[wid], idx_t)
        pltpu.sync_copy(ph_hbm.at[wid], par_h)
        pltpu.sync_copy(pr_hbm.at[wid], par_r)
        pltpu.sync_copy(pt_hbm.at[wid], par_t)

        hb = (hb0, hb1)
        rb = (rb0, rb1)
        tb = (tb0, tb1)
        sems = (sem0, sem1)

        def fire(c):
            slot = c % 2
            return (
                pltpu.async_copy(ent_hbm.at[idx_h.at[c]], hb[slot], sems[slot]),
                pltpu.async_copy(rel_hbm.at[idx_r.at[c]], rb[slot], sems[slot]),
                pltpu.async_copy(ent_hbm.at[idx_t.at[c]], tb[slot], sems[slot]),
            )

        vregs_per_half = dim // _LANES
        lane = lax.broadcasted_iota(jnp.int32, (_LANES,), 0)

        def compute_chunk(c):
            slot = c % 2
            hbuf, rbuf, tbuf = hb[slot], rb[slot], tb[slot]

            def group_body(g, carry):
                base = g * _LANES
                gsl = pl.ds(base, _LANES)
                pvh = par_h[c, gsl]
                pvr = par_r[c, gsl]
                pvt = par_t[c, gsl]

                def triple_body(l, s):
                    lrow = base + l
                    mh = _lane_broadcast(pvh, l) > 0
                    mr = _lane_broadcast(pvr, l) > 0
                    mt = _lane_broadcast(pvt, l) > 0
                    acc = None
                    for j in range(vregs_per_half):
                        lo = pl.ds(j * _LANES, _LANES)
                        hi = pl.ds(dim + j * _LANES, _LANES)
                        hv = jnp.where(mh, hbuf[lrow, hi], hbuf[lrow, lo])
                        rv = jnp.where(mr, rbuf[lrow, hi], rbuf[lrow, lo])
                        tv = jnp.where(mt, tbuf[lrow, hi], tbuf[lrow, lo])
                        d = hv + rv - tv
                        sq = d * d
                        acc = sq if acc is None else acc + sq
                    return jnp.where(lane == l, jnp.sum(acc), s)

                s = lax.fori_loop(0, _LANES, triple_body,
                                  jnp.zeros((_LANES,), jnp.float32))
                outv[pl.ds(c * _CHUNK + base, _LANES)] = _neg_sqrt(s)
                return carry

            lax.fori_loop(0, groups, group_body, 0)

        descs = fire(0)
        for c in range(chunks):
            nxt = fire(c + 1) if c + 1 < chunks else None
            for d in descs:
                d.wait()
            compute_chunk(c)
            descs = nxt

        pltpu.sync_copy(outv, out_hbm.at[pl.ds(wid * b_per_w, b_per_w)])

    return sc_kernel


def _repack_block(tin_ref, out_ref):
    dim = tin_ref.shape[0]
    bc = tin_ref.shape[1] // 2
    blk = tin_ref[...]
    eye = (lax.broadcasted_iota(jnp.int32, (dim, dim), 0)
           == lax.broadcasted_iota(jnp.int32, (dim, dim), 1)
           ).astype(jnp.float32)
    t = lax.dot_general(blk, eye, (((0,), (0,)), ((), ())),
                        preferred_element_type=jnp.float32)
    out_ref[:, 0:dim] = t[0:bc, :]
    out_ref[:, dim:2 * dim] = t[bc:2 * bc, :]


def _repack(table_t, bc):
    dim, n = table_t.shape
    nb = -(-n // (2 * bc))
    return pl.pallas_call(
        _repack_block,
        grid=(nb,),
        in_specs=[pl.BlockSpec((dim, 2 * bc), lambda c: (0, c))],
        out_specs=pl.BlockSpec((bc, 2 * dim), lambda c: (c, 0)),
        out_shape=jax.ShapeDtypeStruct((nb * bc, 2 * dim), jnp.float32),
    )(table_t)


_BC = 8192


def kernel(heads, relations, tails, entity_emb, relation_emb):
    batch = heads.shape[0]
    dim = entity_emb.shape[1]
    chunks = batch // (_NW * _CHUNK)

    ent2 = _repack(entity_emb.T, _BC)
    rel2 = _repack(relation_emb.T, _BC)

    def split(ix):
        ix = ix.astype(jnp.int32).reshape(_NW, chunks, _CHUNK)
        return ((ix >> 14) << 13) + (ix & (_BC - 1)), (ix >> 13) & 1

    hh, ph = split(heads)
    rh, pr = split(relations)
    th, pt = split(tails)

    out = _make_sc_kernel(batch, dim, chunks)(
        hh, rh, th, ph, pr, pt, ent2, rel2)
    return out.reshape(batch, 1)


# speedup vs baseline: 2.8081x; 1.0385x over previous
import functools

import jax
import jax.numpy as jnp
from jax import lax
from jax.experimental import pallas as pl
from jax.experimental.pallas import tpu as pltpu
from jax.experimental.pallas import tpu_sc as plsc

_NC = 2
_NS = 16
_NW = _NC * _NS
_LANES = 16
_CHUNK = 128


_GATHER_DNUMS = lax.GatherDimensionNumbers(
    offset_dims=(), collapsed_slice_dims=(0,), start_index_map=(0,))


def _lane_broadcast(v, l):
    ids = jnp.full((_LANES, 1), l, jnp.int32)
    return lax.gather(v, ids, _GATHER_DNUMS, slice_sizes=(1,),
                      mode=lax.GatherScatterMode.PROMISE_IN_BOUNDS)


def _neg_sqrt(s):
    bits = plsc.bitcast(s, jnp.int32)
    r = plsc.bitcast(jnp.int32(0x5F3759DF) - (bits >> 1), jnp.float32)
    half_s = 0.5 * s
    for _ in range(3):
        r = r * (1.5 - half_s * r * r)
    return -(s * r)


def _make_sc_kernel(batch, dim, chunks):
    b_per_w = chunks * _CHUNK
    groups = _CHUNK // _LANES

    mesh = plsc.VectorSubcoreMesh(core_axis_name="c", subcore_axis_name="s")

    @functools.partial(
        pl.kernel,
        mesh=mesh,
        compiler_params=pltpu.CompilerParams(
            needs_layout_passes=False, use_tc_tiling_on_sc=True),
        out_type=jax.ShapeDtypeStruct((batch,), jnp.float32),
        scratch_types=[
            pltpu.VMEM((chunks, _CHUNK), jnp.int32),
            pltpu.VMEM((chunks, _CHUNK), jnp.int32),
            pltpu.VMEM((chunks, _CHUNK), jnp.int32),
            pltpu.VMEM((chunks, _CHUNK), jnp.int32),
            pltpu.VMEM((chunks, _CHUNK), jnp.int32),
            pltpu.VMEM((chunks, _CHUNK), jnp.int32),
            pltpu.VMEM((_CHUNK, 2 * dim), jnp.float32),
            pltpu.VMEM((_CHUNK, 2 * dim), jnp.float32),
            pltpu.VMEM((_CHUNK, 2 * dim), jnp.float32),
            pltpu.VMEM((_CHUNK, 2 * dim), jnp.float32),
            pltpu.VMEM((_CHUNK, 2 * dim), jnp.float32),
            pltpu.VMEM((_CHUNK, 2 * dim), jnp.float32),
            pltpu.VMEM((b_per_w,), jnp.float32),
            pltpu.SemaphoreType.DMA,
            pltpu.SemaphoreType.DMA,
        ],
    )
    def sc_kernel(heads_hbm, rels_hbm, tails_hbm, ph_hbm, pr_hbm, pt_hbm,
                  ent_hbm, rel_hbm, out_hbm,
                  idx_h, idx_r, idx_t, par_h, par_r, par_t,
                  hb0, hb1, rb0, rb1, tb0, tb1, outv, sem0, sem1):
        wid = lax.axis_index("s") * _NC + lax.axis_index("c")

        pltpu.sync_copy(heads_hbm.at[wid], idx_h)
        pltpu.sync_copy(rels_hbm.at[wid], idx_r)
        pltpu.sync_copy(tails_hbm.at[wid], idx_t)
        pltpu.sync_copy(ph_hbm.at[wid], par_h)
        pltpu.sync_copy(pr_hbm.at[wid], par_r)
        pltpu.sync_copy(pt_hbm.at[wid], par_t)

        hb = (hb0, hb1)
        rb = (rb0, rb1)
        tb = (tb0, tb1)
        sems = (sem0, sem1)

        def fire(c):
            slot = c % 2
            return (
                pltpu.async_copy(ent_hbm.at[idx_h.at[c]], hb[slot], sems[slot]),
                pltpu.async_copy(rel_hbm.at[idx_r.at[c]], rb[slot], sems[slot]),
                pltpu.async_copy(ent_hbm.at[idx_t.at[c]], tb[slot], sems[slot]),
            )

        vregs_per_half = dim // _LANES
        lane = lax.broadcasted_iota(jnp.int32, (_LANES,), 0)

        def compute_chunk(c):
            slot = c % 2
            hbuf, rbuf, tbuf = hb[slot], rb[slot], tb[slot]

            def group_body(g, carry):
                base = g * _LANES
                gsl = pl.ds(base, _LANES)
                pvh = par_h[c, gsl]
                pvr = par_r[c, gsl]
                pvt = par_t[c, gsl]

                def triple_body(l, s):
                    lrow = base + l
                    mh = _lane_broadcast(pvh, l) > 0
                    mr = _lane_broadcast(pvr, l) > 0
                    mt = _lane_broadcast(pvt, l) > 0
                    acc = None
                    for j in range(vregs_per_half):
                        lo = pl.ds(j * _LANES, _LANES)
                        hi = pl.ds(dim + j * _LANES, _LANES)
                        hv = jnp.where(mh, hbuf[lrow, hi], hbuf[lrow, lo])
                        rv = jnp.where(mr, rbuf[lrow, hi], rbuf[lrow, lo])
                        tv = jnp.where(mt, tbuf[lrow, hi], tbuf[lrow, lo])
                        d = hv + rv - tv
                        sq = d * d
                        acc = sq if acc is None else acc + sq
                    return jnp.where(lane == l, jnp.sum(acc), s)

                s = lax.fori_loop(0, _LANES, triple_body,
                                  jnp.zeros((_LANES,), jnp.float32))
                outv[pl.ds(c * _CHUNK + base, _LANES)] = _neg_sqrt(s)
                return carry

            lax.fori_loop(0, groups, group_body, 0)

        descs = fire(0)
        for c in range(chunks):
            nxt = fire(c + 1) if c + 1 < chunks else None
            for d in descs:
                d.wait()
            compute_chunk(c)
            descs = nxt

        pltpu.sync_copy(outv, out_hbm.at[pl.ds(wid * b_per_w, b_per_w)])

    return sc_kernel


def _repack_block(tin_ref, out_ref):
    dim = tin_ref.shape[0]
    bc = tin_ref.shape[1] // 2
    blk = tin_ref[...]
    eye = (lax.broadcasted_iota(jnp.int32, (dim, dim), 0)
           == lax.broadcasted_iota(jnp.int32, (dim, dim), 1)
           ).astype(jnp.float32)
    t = lax.dot_general(blk, eye, (((0,), (0,)), ((), ())),
                        preferred_element_type=jnp.float32)
    out_ref[:, 0:dim] = t[0:bc, :]
    out_ref[:, dim:2 * dim] = t[bc:2 * bc, :]


def _repack(table_t, bc):
    dim, n = table_t.shape
    nb = -(-n // (2 * bc))
    return pl.pallas_call(
        _repack_block,
        grid=(nb,),
        in_specs=[pl.BlockSpec((dim, 2 * bc), lambda c: (0, c))],
        out_specs=pl.BlockSpec((bc, 2 * dim), lambda c: (c, 0)),
        out_shape=jax.ShapeDtypeStruct((nb * bc, 2 * dim), jnp.float32),
    )(table_t)


_BC = 16384


def kernel(heads, relations, tails, entity_emb, relation_emb):
    batch = heads.shape[0]
    dim = entity_emb.shape[1]
    chunks = batch // (_NW * _CHUNK)

    ent2 = _repack(entity_emb.T, _BC)
    rel2 = _repack(relation_emb.T, _BC)

    def split(ix):
        ix = ix.astype(jnp.int32).reshape(_NW, chunks, _CHUNK)
        return ((ix >> 15) << 14) + (ix & (_BC - 1)), (ix >> 14) & 1

    hh, ph = split(heads)
    rh, pr = split(relations)
    th, pt = split(tails)

    out = _make_sc_kernel(batch, dim, chunks)(
        hh, rh, th, ph, pr, pt, ent2, rel2)
    return out.reshape(batch, 1)


# speedup vs baseline: 2.8780x; 1.0249x over previous
import functools

import jax
import jax.numpy as jnp
from jax import lax
from jax.experimental import pallas as pl
from jax.experimental.pallas import tpu as pltpu
from jax.experimental.pallas import tpu_sc as plsc

_NC = 2
_NS = 16
_NW = _NC * _NS
_LANES = 16
_CHUNK = 128


_GATHER_DNUMS = lax.GatherDimensionNumbers(
    offset_dims=(), collapsed_slice_dims=(0,), start_index_map=(0,))


def _lane_broadcast(v, l):
    ids = jnp.full((_LANES, 1), l, jnp.int32)
    return lax.gather(v, ids, _GATHER_DNUMS, slice_sizes=(1,),
                      mode=lax.GatherScatterMode.PROMISE_IN_BOUNDS)


def _neg_sqrt(s):
    bits = plsc.bitcast(s, jnp.int32)
    r = plsc.bitcast(jnp.int32(0x5F3759DF) - (bits >> 1), jnp.float32)
    half_s = 0.5 * s
    for _ in range(3):
        r = r * (1.5 - half_s * r * r)
    return -(s * r)


def _make_sc_kernel(batch, dim, chunks):
    b_per_w = chunks * _CHUNK
    groups = _CHUNK // _LANES

    mesh = plsc.VectorSubcoreMesh(core_axis_name="c", subcore_axis_name="s")

    @functools.partial(
        pl.kernel,
        mesh=mesh,
        compiler_params=pltpu.CompilerParams(
            needs_layout_passes=False, use_tc_tiling_on_sc=True),
        out_type=jax.ShapeDtypeStruct((batch,), jnp.float32),
        scratch_types=[
            pltpu.VMEM((chunks, _CHUNK), jnp.int32),
            pltpu.VMEM((chunks, _CHUNK), jnp.int32),
            pltpu.VMEM((chunks, _CHUNK), jnp.int32),
            pltpu.VMEM((chunks, _CHUNK), jnp.int32),
            pltpu.VMEM((chunks, _CHUNK), jnp.int32),
            pltpu.VMEM((chunks, _CHUNK), jnp.int32),
            pltpu.VMEM((_CHUNK, 2 * dim), jnp.float32),
            pltpu.VMEM((_CHUNK, 2 * dim), jnp.float32),
            pltpu.VMEM((_CHUNK, 2 * dim), jnp.float32),
            pltpu.VMEM((_CHUNK, 2 * dim), jnp.float32),
            pltpu.VMEM((_CHUNK, 2 * dim), jnp.float32),
            pltpu.VMEM((_CHUNK, 2 * dim), jnp.float32),
            pltpu.VMEM((b_per_w,), jnp.float32),
            pltpu.SemaphoreType.DMA,
            pltpu.SemaphoreType.DMA,
        ],
    )
    def sc_kernel(heads_hbm, rels_hbm, tails_hbm, ph_hbm, pr_hbm, pt_hbm,
                  ent_hbm, rel_hbm, out_hbm,
                  idx_h, idx_r, idx_t, par_h, par_r, par_t,
                  hb0, hb1, rb0, rb1, tb0, tb1, outv, sem0, sem1):
        wid = lax.axis_index("s") * _NC + lax.axis_index("c")

        pltpu.sync_copy(heads_hbm.at[wid], idx_h)
        pltpu.sync_copy(rels_hbm.at[wid], idx_r)
        pltpu.sync_copy(tails_hbm.at[wid], idx_t)
        pltpu.sync_copy(ph_hbm.at[wid], par_h)
        pltpu.sync_copy(pr_hbm.at[wid], par_r)
        pltpu.sync_copy(pt_hbm.at[wid], par_t)

        hb = (hb0, hb1)
        rb = (rb0, rb1)
        tb = (tb0, tb1)
        sems = (sem0, sem1)

        def fire(c):
            slot = c % 2
            return (
                pltpu.async_copy(ent_hbm.at[idx_h.at[c]], hb[slot], sems[slot]),
                pltpu.async_copy(rel_hbm.at[idx_r.at[c]], rb[slot], sems[slot]),
                pltpu.async_copy(ent_hbm.at[idx_t.at[c]], tb[slot], sems[slot]),
            )

        vregs_per_half = dim // _LANES
        lane = lax.broadcasted_iota(jnp.int32, (_LANES,), 0)

        def compute_chunk(c):
            slot = c % 2
            hbuf, rbuf, tbuf = hb[slot], rb[slot], tb[slot]

            def group_body(g, carry):
                base = g * _LANES
                gsl = pl.ds(base, _LANES)
                pvh = par_h[c, gsl]
                pvr = par_r[c, gsl]
                pvt = par_t[c, gsl]

                def triple_body(l, s):
                    lrow = base + l
                    mh = _lane_broadcast(pvh, l) > 0
                    mr = _lane_broadcast(pvr, l) > 0
                    mt = _lane_broadcast(pvt, l) > 0
                    acc = None
                    for j in range(vregs_per_half):
                        lo = pl.ds(j * _LANES, _LANES)
                        hi = pl.ds(dim + j * _LANES, _LANES)
                        hv = jnp.where(mh, hbuf[lrow, hi], hbuf[lrow, lo])
                        rv = jnp.where(mr, rbuf[lrow, hi], rbuf[lrow, lo])
                        tv = jnp.where(mt, tbuf[lrow, hi], tbuf[lrow, lo])
                        d = hv + rv - tv
                        sq = d * d
                        acc = sq if acc is None else acc + sq
                    return jnp.where(lane == l, jnp.sum(acc), s)

                s = lax.fori_loop(0, _LANES, triple_body,
                                  jnp.zeros((_LANES,), jnp.float32))
                outv[pl.ds(c * _CHUNK + base, _LANES)] = _neg_sqrt(s)
                return carry

            lax.fori_loop(0, groups, group_body, 0)

        descs = fire(0)
        for c in range(chunks):
            nxt = fire(c + 1) if c + 1 < chunks else None
            for d in descs:
                d.wait()
            compute_chunk(c)
            descs = nxt

        pltpu.sync_copy(outv, out_hbm.at[pl.ds(wid * b_per_w, b_per_w)])

    return sc_kernel


def _mxu_transpose(blk, dim):
    eye = (lax.broadcasted_iota(jnp.int32, (dim, dim), 0)
           == lax.broadcasted_iota(jnp.int32, (dim, dim), 1)
           ).astype(jnp.float32)
    return lax.dot_general(blk, eye, (((0,), (0,)), ((), ())),
                           preferred_element_type=jnp.float32)


def _repack_block(tin_ref, rin_ref, out_ref, rout_ref):
    dim = tin_ref.shape[0]
    bc = tin_ref.shape[1] // 2
    t = _mxu_transpose(tin_ref[...], dim)
    out_ref[:, 0:dim] = t[0:bc, :]
    out_ref[:, dim:2 * dim] = t[bc:2 * bc, :]
    nrel = rin_ref.shape[1]
    rout_ref[0:nrel, 0:dim] = _mxu_transpose(rin_ref[...], dim)


def _repack(table_t, rel_t, bc):
    dim, n = table_t.shape
    nrel = rel_t.shape[1]
    assert nrel <= bc
    nb = -(-n // (2 * bc))
    return pl.pallas_call(
        _repack_block,
        grid=(nb,),
        in_specs=[pl.BlockSpec((dim, 2 * bc), lambda c: (0, c)),
                  pl.BlockSpec((dim, nrel), lambda c: (0, 0))],
        out_specs=[pl.BlockSpec((bc, 2 * dim), lambda c: (c, 0)),
                   pl.BlockSpec((bc, 2 * dim), lambda c: (0, 0))],
        out_shape=[jax.ShapeDtypeStruct((nb * bc, 2 * dim), jnp.float32),
                   jax.ShapeDtypeStruct((bc, 2 * dim), jnp.float32)],
    )(table_t, rel_t)


_BC = 16384


def kernel(heads, relations, tails, entity_emb, relation_emb):
    batch = heads.shape[0]
    dim = entity_emb.shape[1]
    chunks = batch // (_NW * _CHUNK)

    ent2, rel2 = _repack(entity_emb.T, relation_emb.T, _BC)

    def split(ix):
        ix = ix.astype(jnp.int32).reshape(_NW, chunks, _CHUNK)
        return ((ix >> 15) << 14) + (ix & (_BC - 1)), (ix >> 14) & 1

    hh, ph = split(heads)
    rh, pr = split(relations)
    th, pt = split(tails)

    out = _make_sc_kernel(batch, dim, chunks)(
        hh, rh, th, ph, pr, pt, ent2, rel2)
    return out.reshape(batch, 1)


# speedup vs baseline: 2.9314x; 1.0185x over previous
import functools

import jax
import jax.numpy as jnp
from jax import lax
from jax.experimental import pallas as pl
from jax.experimental.pallas import tpu as pltpu
from jax.experimental.pallas import tpu_sc as plsc

_NC = 2
_NS = 16
_NW = _NC * _NS
_LANES = 16
_CHUNK = 128


_GATHER_DNUMS = lax.GatherDimensionNumbers(
    offset_dims=(), collapsed_slice_dims=(0,), start_index_map=(0,))


def _lane_broadcast(v, l):
    ids = jnp.full((_LANES, 1), l, jnp.int32)
    return lax.gather(v, ids, _GATHER_DNUMS, slice_sizes=(1,),
                      mode=lax.GatherScatterMode.PROMISE_IN_BOUNDS)


def _neg_sqrt(s):
    bits = plsc.bitcast(s, jnp.int32)
    r = plsc.bitcast(jnp.int32(0x5F3759DF) - (bits >> 1), jnp.float32)
    half_s = 0.5 * s
    for _ in range(3):
        r = r * (1.5 - half_s * r * r)
    return -(s * r)


def _make_sc_kernel(batch, dim, chunks):
    b_per_w = chunks * _CHUNK
    groups = _CHUNK // _LANES

    mesh = plsc.VectorSubcoreMesh(core_axis_name="c", subcore_axis_name="s")

    @functools.partial(
        pl.kernel,
        mesh=mesh,
        compiler_params=pltpu.CompilerParams(
            needs_layout_passes=False, use_tc_tiling_on_sc=True),
        out_type=jax.ShapeDtypeStruct((batch,), jnp.float32),
        scratch_types=[
            pltpu.VMEM((chunks, _CHUNK), jnp.int32),
            pltpu.VMEM((chunks, _CHUNK), jnp.int32),
            pltpu.VMEM((chunks, _CHUNK), jnp.int32),
            pltpu.VMEM((chunks, _CHUNK), jnp.int32),
            pltpu.VMEM((chunks, _CHUNK), jnp.int32),
            pltpu.VMEM((chunks, _CHUNK), jnp.int32),
            pltpu.VMEM((_CHUNK, 2 * dim), jnp.float32),
            pltpu.VMEM((_CHUNK, 2 * dim), jnp.float32),
            pltpu.VMEM((_CHUNK, 2 * dim), jnp.float32),
            pltpu.VMEM((_CHUNK, 2 * dim), jnp.float32),
            pltpu.VMEM((_CHUNK, 2 * dim), jnp.float32),
            pltpu.VMEM((_CHUNK, 2 * dim), jnp.float32),
            pltpu.VMEM((b_per_w,), jnp.float32),
            pltpu.SemaphoreType.DMA,
            pltpu.SemaphoreType.DMA,
        ],
    )
    def sc_kernel(heads_hbm, rels_hbm, tails_hbm,
                  ent_hbm, rel_hbm, out_hbm,
                  idx_h, idx_r, idx_t, par_h, par_r, par_t,
                  hb0, hb1, rb0, rb1, tb0, tb1, outv, sem0, sem1):
        wid = lax.axis_index("s") * _NC + lax.axis_index("c")

        pltpu.sync_copy(heads_hbm.at[wid], idx_h)
        pltpu.sync_copy(rels_hbm.at[wid], idx_r)
        pltpu.sync_copy(tails_hbm.at[wid], idx_t)
        for idx, par in ((idx_h, par_h), (idx_r, par_r), (idx_t, par_t)):
            for c in range(chunks):
                for g in range(_CHUNK // _LANES):
                    sl = pl.ds(g * _LANES, _LANES)
                    v = idx[c, sl]
                    par[c, sl] = (v >> 14) & 1
                    idx[c, sl] = ((v >> 15) << 14) + (v & 16383)

        hb = (hb0, hb1)
        rb = (rb0, rb1)
        tb = (tb0, tb1)
        sems = (sem0, sem1)

        def fire(c):
            slot = c % 2
            return (
                pltpu.async_copy(ent_hbm.at[idx_h.at[c]], hb[slot], sems[slot]),
                pltpu.async_copy(rel_hbm.at[idx_r.at[c]], rb[slot], sems[slot]),
                pltpu.async_copy(ent_hbm.at[idx_t.at[c]], tb[slot], sems[slot]),
            )

        vregs_per_half = dim // _LANES
        lane = lax.broadcasted_iota(jnp.int32, (_LANES,), 0)

        def compute_chunk(c):
            slot = c % 2
            hbuf, rbuf, tbuf = hb[slot], rb[slot], tb[slot]

            def group_body(g, carry):
                base = g * _LANES
                gsl = pl.ds(base, _LANES)
                pvh = par_h[c, gsl]
                pvr = par_r[c, gsl]
                pvt = par_t[c, gsl]

                def triple_body(l, s):
                    lrow = base + l
                    mh = _lane_broadcast(pvh, l) > 0
                    mr = _lane_broadcast(pvr, l) > 0
                    mt = _lane_broadcast(pvt, l) > 0
                    acc = None
                    for j in range(vregs_per_half):
                        lo = pl.ds(j * _LANES, _LANES)
                        hi = pl.ds(dim + j * _LANES, _LANES)
                        hv = jnp.where(mh, hbuf[lrow, hi], hbuf[lrow, lo])
                        rv = jnp.where(mr, rbuf[lrow, hi], rbuf[lrow, lo])
                        tv = jnp.where(mt, tbuf[lrow, hi], tbuf[lrow, lo])
                        d = hv + rv - tv
                        sq = d * d
                        acc = sq if acc is None else acc + sq
                    return jnp.where(lane == l, jnp.sum(acc), s)

                s = lax.fori_loop(0, _LANES, triple_body,
                                  jnp.zeros((_LANES,), jnp.float32))
                outv[pl.ds(c * _CHUNK + base, _LANES)] = _neg_sqrt(s)
                return carry

            lax.fori_loop(0, groups, group_body, 0)

        descs = fire(0)
        for c in range(chunks):
            nxt = fire(c + 1) if c + 1 < chunks else None
            for d in descs:
                d.wait()
            compute_chunk(c)
            descs = nxt

        pltpu.sync_copy(outv, out_hbm.at[pl.ds(wid * b_per_w, b_per_w)])

    return sc_kernel


def _mxu_transpose(blk, dim):
    eye = (lax.broadcasted_iota(jnp.int32, (dim, dim), 0)
           == lax.broadcasted_iota(jnp.int32, (dim, dim), 1)
           ).astype(jnp.float32)
    return lax.dot_general(blk, eye, (((0,), (0,)), ((), ())),
                           preferred_element_type=jnp.float32)


def _repack_block(tin_ref, rin_ref, out_ref, rout_ref):
    dim = tin_ref.shape[0]
    bc = tin_ref.shape[1] // 2
    t = _mxu_transpose(tin_ref[...], dim)
    out_ref[:, 0:dim] = t[0:bc, :]
    out_ref[:, dim:2 * dim] = t[bc:2 * bc, :]
    nrel = rin_ref.shape[1]
    rout_ref[0:nrel, 0:dim] = _mxu_transpose(rin_ref[...], dim)


def _repack(table_t, rel_t, bc):
    dim, n = table_t.shape
    nrel = rel_t.shape[1]
    assert nrel <= bc
    nb = -(-n // (2 * bc))
    return pl.pallas_call(
        _repack_block,
        grid=(nb,),
        in_specs=[pl.BlockSpec((dim, 2 * bc), lambda c: (0, c)),
                  pl.BlockSpec((dim, nrel), lambda c: (0, 0))],
        out_specs=[pl.BlockSpec((bc, 2 * dim), lambda c: (c, 0)),
                   pl.BlockSpec((bc, 2 * dim), lambda c: (0, 0))],
        out_shape=[jax.ShapeDtypeStruct((nb * bc, 2 * dim), jnp.float32),
                   jax.ShapeDtypeStruct((bc, 2 * dim), jnp.float32)],
    )(table_t, rel_t)


_BC = 16384


def kernel(heads, relations, tails, entity_emb, relation_emb):
    batch = heads.shape[0]
    dim = entity_emb.shape[1]
    chunks = batch // (_NW * _CHUNK)

    ent2, rel2 = _repack(entity_emb.T, relation_emb.T, _BC)

    hh = heads.astype(jnp.int32).reshape(_NW, chunks, _CHUNK)
    rh = relations.astype(jnp.int32).reshape(_NW, chunks, _CHUNK)
    th = tails.astype(jnp.int32).reshape(_NW, chunks, _CHUNK)

    out = _make_sc_kernel(batch, dim, chunks)(
        hh, rh, th, ent2, rel2)
    return out.reshape(batch, 1)
